# Initial kernel scaffold; baseline (speedup 1.0000x reference)
import jax
import jax.numpy as jnp
from jax.experimental import pallas as pl


def kernel(node_emb, step, detach_weights):
    raise NotImplementedError("write your pallas kernel here")


# speedup vs baseline: 3.7015x; 3.7015x over previous
import functools

import jax
import jax.numpy as jnp
from jax.experimental import pallas as pl

_N = 10000
_D = 128
_K = 16
_TEMP = 0.07
_EPS = 1e-8
_ROWS = 200
_CPAD = 10112


def _topk_softmax_body(n, k, temp, rows, cpad, e_ref, et_ref, w_ref, idx_ref):
    pid = pl.program_id(0)
    sim = jnp.dot(e_ref[...], et_ref[...], preferred_element_type=jnp.float32)
    row_g = pid * rows + jax.lax.broadcasted_iota(jnp.int32, (rows, cpad), 0)
    col = jax.lax.broadcasted_iota(jnp.int32, (rows, cpad), 1)
    sim = jnp.where((col == row_g) | (col >= n), -1e9, sim)

    lane = jax.lax.broadcasted_iota(jnp.int32, (rows, k), 1)
    vals = jnp.zeros((rows, k), jnp.float32)
    idxs = jnp.zeros((rows, k), jnp.int32)
    for j in range(k):
        m = jnp.max(sim, axis=1)
        eq = sim == m[:, None]
        a = jnp.min(jnp.where(eq, col, cpad), axis=1)
        sim = jnp.where(col == a[:, None], -jnp.inf, sim)
        vals = jnp.where(lane == j, m[:, None], vals)
        idxs = jnp.where(lane == j, a[:, None], idxs)

    ex = jnp.exp((vals - vals[:, 0][:, None]) / temp)
    s = jnp.sum(ex, axis=1, keepdims=True)
    w_ref[...] = ex / s * (k / (k + 1.0))
    idx_ref[...] = idxs


def _graph_weights(e):
    et = jnp.zeros((_D, _CPAD), jnp.float32).at[:, :_N].set(e.T)
    grid = _N // _ROWS
    return pl.pallas_call(
        functools.partial(_topk_softmax_body, _N, _K, _TEMP, _ROWS, _CPAD),
        grid=(grid,),
        in_specs=[
            pl.BlockSpec((_ROWS, _D), lambda i: (i, 0)),
            pl.BlockSpec((_D, _CPAD), lambda i: (0, 0)),
        ],
        out_specs=[
            pl.BlockSpec((_ROWS, _K), lambda i: (i, 0)),
            pl.BlockSpec((_ROWS, _K), lambda i: (i, 0)),
        ],
        out_shape=[
            jax.ShapeDtypeStruct((_N, _K), jnp.float32),
            jax.ShapeDtypeStruct((_N, _K), jnp.int32),
        ],
    )(e, et)


def kernel(node_emb, step, detach_weights):
    nrm = jnp.sqrt(jnp.sum(node_emb * node_emb, axis=1, keepdims=True))
    e = node_emb / jnp.maximum(nrm, _EPS)
    w, knn_idx = _graph_weights(e)

    row = jnp.repeat(jnp.arange(_N, dtype=jnp.int64), _K)
    col = knn_idx.reshape(-1).astype(jnp.int64)
    val = w.reshape(-1)

    self_row = jnp.arange(_N, dtype=jnp.int64)
    self_val = jnp.full((_N,), 1.0 / (_K + 1), dtype=val.dtype)
    row = jnp.concatenate([row, self_row], axis=0)
    col = jnp.concatenate([col, self_row], axis=0)
    val = jnp.concatenate([val, self_val], axis=0)

    row0, col0, val0 = row, col, val
    row = jnp.concatenate([row0, col0], axis=0)
    col = jnp.concatenate([col0, row0], axis=0)
    val = jnp.concatenate([val0, val0], axis=0)

    indices = jnp.stack([row, col], axis=0)
    return indices, val


# speedup vs baseline: 3.7017x; 1.0000x over previous
import functools

import jax
import jax.numpy as jnp
from jax.experimental import pallas as pl
from jax.experimental.pallas import tpu as pltpu

_N = 10000
_D = 128
_K = 16
_TEMP = 0.07
_EPS = 1e-8
_ROWS = 200
_CPAD = 10112


def _topk_softmax_body(n, k, temp, rows, cpad, e_ref, et_ref, w_ref, idx_ref):
    pid = pl.program_id(0)
    sim = jnp.dot(e_ref[...], et_ref[...], preferred_element_type=jnp.float32)
    row_g = pid * rows + jax.lax.broadcasted_iota(jnp.int32, (rows, cpad), 0)
    col = jax.lax.broadcasted_iota(jnp.int32, (rows, cpad), 1)
    sim = jnp.where((col == row_g) | (col >= n), -1e9, sim)

    lane = jax.lax.broadcasted_iota(jnp.int32, (rows, k), 1)
    vals = jnp.zeros((rows, k), jnp.float32)
    idxs = jnp.zeros((rows, k), jnp.int32)
    m = jnp.max(sim, axis=1)
    for j in range(k):
        a = jnp.min(jnp.where(sim == m[:, None], col, cpad), axis=1)
        vals = jnp.where(lane == j, m[:, None], vals)
        idxs = jnp.where(lane == j, a[:, None], idxs)
        if j < k - 1:
            sim = jnp.where(col == a[:, None], -jnp.inf, sim)
            m = jnp.max(sim, axis=1)

    ex = jnp.exp((vals - vals[:, 0][:, None]) / temp)
    s = jnp.sum(ex, axis=1, keepdims=True)
    w_ref[...] = ex / s * (k / (k + 1.0))
    idx_ref[...] = idxs


def _graph_weights(e):
    et = jnp.zeros((_D, _CPAD), jnp.float32).at[:, :_N].set(e.T)
    grid = _N // _ROWS
    return pl.pallas_call(
        functools.partial(_topk_softmax_body, _N, _K, _TEMP, _ROWS, _CPAD),
        grid=(grid,),
        in_specs=[
            pl.BlockSpec((_ROWS, _D), lambda i: (i, 0)),
            pl.BlockSpec((_D, _CPAD), lambda i: (0, 0)),
        ],
        out_specs=[
            pl.BlockSpec((_ROWS, _K), lambda i: (i, 0)),
            pl.BlockSpec((_ROWS, _K), lambda i: (i, 0)),
        ],
        out_shape=[
            jax.ShapeDtypeStruct((_N, _K), jnp.float32),
            jax.ShapeDtypeStruct((_N, _K), jnp.int32),
        ],
        compiler_params=pltpu.CompilerParams(
            dimension_semantics=("parallel",),
        ),
    )(e, et)


def kernel(node_emb, step, detach_weights):
    nrm = jnp.sqrt(jnp.sum(node_emb * node_emb, axis=1, keepdims=True))
    e = node_emb / jnp.maximum(nrm, _EPS)
    w, knn_idx = _graph_weights(e)

    row = jnp.repeat(jnp.arange(_N, dtype=jnp.int64), _K)
    col = knn_idx.reshape(-1).astype(jnp.int64)
    val = w.reshape(-1)

    self_row = jnp.arange(_N, dtype=jnp.int64)
    self_val = jnp.full((_N,), 1.0 / (_K + 1), dtype=val.dtype)
    row = jnp.concatenate([row, self_row], axis=0)
    col = jnp.concatenate([col, self_row], axis=0)
    val = jnp.concatenate([val, self_val], axis=0)

    row0, col0, val0 = row, col, val
    row = jnp.concatenate([row0, col0], axis=0)
    col = jnp.concatenate([col0, row0], axis=0)
    val = jnp.concatenate([val0, val0], axis=0)

    indices = jnp.stack([row, col], axis=0)
    return indices, val


# speedup vs baseline: 8.6199x; 2.3286x over previous
import functools

import jax
import jax.numpy as jnp
from jax.experimental import pallas as pl
from jax.experimental.pallas import tpu as pltpu

_N = 10000
_D = 128
_K = 16
_TEMP = 0.07
_EPS = 1e-8
_ROWS = 200
_CPAD = 10240
_SLOTS = 4


def _store_topk(temp, k, rows, vals, idxs, w_ref, idx_ref):
    ex = jnp.exp((vals - vals[:, 0][:, None]) / temp)
    s = jnp.sum(ex, axis=1, keepdims=True)
    w_ref[...] = ex / s * (k / (k + 1.0))
    idx_ref[...] = idxs


def _topk_softmax_body(n, k, temp, rows, cpad, slots, e_ref, et_ref, w_ref,
                       idx_ref):
    pid = pl.program_id(0)
    sim = jnp.dot(e_ref[...], et_ref[...], preferred_element_type=jnp.float32)
    row_g = pid * rows + jax.lax.broadcasted_iota(jnp.int32, (rows, cpad), 0)
    col = jax.lax.broadcasted_iota(jnp.int32, (rows, cpad), 1)
    sim = jnp.where((col == row_g) | (col >= n), -1e9, sim)

    nb = cpad // 128
    lane128 = jax.lax.broadcasted_iota(jnp.int32, (rows, 128), 1)
    neg = jnp.full((rows, 128), -jnp.inf, jnp.float32)
    rv = [neg for _ in range(slots)]
    rb = [jnp.zeros((rows, 128), jnp.int32) for _ in range(slots)]
    spill = neg
    for b in range(nb):
        v = sim[:, b * 128:(b + 1) * 128]
        bb = jnp.full((rows, 128), b, jnp.int32)
        for s in range(slots):
            keep = rv[s] >= v
            nv = jnp.where(keep, rv[s], v)
            nbk = jnp.where(keep, rb[s], bb)
            v = jnp.where(keep, v, rv[s])
            bb = jnp.where(keep, bb, rb[s])
            rv[s], rb[s] = nv, nbk
        spill = jnp.maximum(spill, v)

    cand_v = jnp.concatenate(rv, axis=1)
    cand_c = jnp.concatenate([r * 128 + lane128 for r in rb], axis=1)

    lane = jax.lax.broadcasted_iota(jnp.int32, (rows, k), 1)
    vals = jnp.zeros((rows, k), jnp.float32)
    idxs = jnp.zeros((rows, k), jnp.int32)
    m = jnp.max(cand_v, axis=1)
    for j in range(k):
        a = jnp.min(jnp.where(cand_v == m[:, None], cand_c, cpad), axis=1)
        vals = jnp.where(lane == j, m[:, None], vals)
        idxs = jnp.where(lane == j, a[:, None], idxs)
        if j < k - 1:
            sel = (cand_v == m[:, None]) & (cand_c == a[:, None])
            cand_v = jnp.where(sel, -jnp.inf, cand_v)
            m = jnp.max(cand_v, axis=1)

    _store_topk(temp, k, rows, vals, idxs, w_ref, idx_ref)

    bad = jnp.any(spill >= vals[:, k - 1][:, None])

    @pl.when(bad)
    def _slow():
        s2 = sim
        vals2 = jnp.zeros((rows, k), jnp.float32)
        idxs2 = jnp.zeros((rows, k), jnp.int32)
        m2 = jnp.max(s2, axis=1)
        for j in range(k):
            a2 = jnp.min(jnp.where(s2 == m2[:, None], col, cpad), axis=1)
            vals2 = jnp.where(lane == j, m2[:, None], vals2)
            idxs2 = jnp.where(lane == j, a2[:, None], idxs2)
            if j < k - 1:
                s3 = jnp.where(col == a2[:, None], -jnp.inf, s2)
                m2 = jnp.max(s3, axis=1)
                s2 = s3
        _store_topk(temp, k, rows, vals2, idxs2, w_ref, idx_ref)


def _graph_weights(e):
    et = jnp.zeros((_D, _CPAD), jnp.float32).at[:, :_N].set(e.T)
    grid = _N // _ROWS
    return pl.pallas_call(
        functools.partial(_topk_softmax_body, _N, _K, _TEMP, _ROWS, _CPAD,
                          _SLOTS),
        grid=(grid,),
        in_specs=[
            pl.BlockSpec((_ROWS, _D), lambda i: (i, 0)),
            pl.BlockSpec((_D, _CPAD), lambda i: (0, 0)),
        ],
        out_specs=[
            pl.BlockSpec((_ROWS, _K), lambda i: (i, 0)),
            pl.BlockSpec((_ROWS, _K), lambda i: (i, 0)),
        ],
        out_shape=[
            jax.ShapeDtypeStruct((_N, _K), jnp.float32),
            jax.ShapeDtypeStruct((_N, _K), jnp.int32),
        ],
        compiler_params=pltpu.CompilerParams(
            dimension_semantics=("arbitrary",),
        ),
    )(e, et)


def kernel(node_emb, step, detach_weights):
    nrm = jnp.sqrt(jnp.sum(node_emb * node_emb, axis=1, keepdims=True))
    e = node_emb / jnp.maximum(nrm, _EPS)
    w, knn_idx = _graph_weights(e)

    row = jnp.repeat(jnp.arange(_N, dtype=jnp.int64), _K)
    col = knn_idx.reshape(-1).astype(jnp.int64)
    val = w.reshape(-1)

    self_row = jnp.arange(_N, dtype=jnp.int64)
    self_val = jnp.full((_N,), 1.0 / (_K + 1), dtype=val.dtype)
    row = jnp.concatenate([row, self_row], axis=0)
    col = jnp.concatenate([col, self_row], axis=0)
    val = jnp.concatenate([val, self_val], axis=0)

    row0, col0, val0 = row, col, val
    row = jnp.concatenate([row0, col0], axis=0)
    col = jnp.concatenate([col0, row0], axis=0)
    val = jnp.concatenate([val0, val0], axis=0)

    indices = jnp.stack([row, col], axis=0)
    return indices, val


# speedup vs baseline: 9.5670x; 1.1099x over previous
import functools

import jax
import jax.numpy as jnp
from jax.experimental import pallas as pl
from jax.experimental.pallas import tpu as pltpu

_N = 10000
_D = 128
_K = 16
_TEMP = 0.07
_EPS = 1e-8
_ROWS = 400
_CPAD = 10240
_SLOTS = 4


def _store_topk(temp, k, rows, vals, idxs, w_ref, idx_ref):
    ex = jnp.exp((vals - vals[:, 0][:, None]) / temp)
    s = jnp.sum(ex, axis=1, keepdims=True)
    w_ref[...] = ex / s * (k / (k + 1.0))
    idx_ref[...] = idxs


def _topk_softmax_body(n, k, temp, rows, cpad, slots, e_ref, et_ref, w_ref,
                       idx_ref):
    pid = pl.program_id(0)
    sim = jnp.dot(e_ref[...], et_ref[...], preferred_element_type=jnp.float32)
    row_g = pid * rows + jax.lax.broadcasted_iota(jnp.int32, (rows, cpad), 0)
    col = jax.lax.broadcasted_iota(jnp.int32, (rows, cpad), 1)
    sim = jnp.where((col == row_g) | (col >= n), -1e9, sim)

    nb = cpad // 128
    lane128 = jax.lax.broadcasted_iota(jnp.int32, (rows, 128), 1)
    neg = jnp.full((rows, 128), -jnp.inf, jnp.float32)
    rv = [neg for _ in range(slots)]
    rb = [jnp.zeros((rows, 128), jnp.int32) for _ in range(slots)]
    spill = neg
    for b in range(nb):
        v = sim[:, b * 128:(b + 1) * 128]
        bb = jnp.full((rows, 128), b, jnp.int32)
        for s in range(slots):
            keep = rv[s] >= v
            nv = jnp.where(keep, rv[s], v)
            nbk = jnp.where(keep, rb[s], bb)
            v = jnp.where(keep, v, rv[s])
            bb = jnp.where(keep, bb, rb[s])
            rv[s], rb[s] = nv, nbk
        spill = jnp.maximum(spill, v)

    cand_v = jnp.concatenate(rv, axis=1)
    cand_c = jnp.concatenate([r * 128 + lane128 for r in rb], axis=1)

    lane = jax.lax.broadcasted_iota(jnp.int32, (rows, k), 1)
    vals = jnp.zeros((rows, k), jnp.float32)
    idxs = jnp.zeros((rows, k), jnp.int32)
    m = jnp.max(cand_v, axis=1)
    for j in range(k):
        a = jnp.min(jnp.where(cand_v == m[:, None], cand_c, cpad), axis=1)
        vals = jnp.where(lane == j, m[:, None], vals)
        idxs = jnp.where(lane == j, a[:, None], idxs)
        if j < k - 1:
            sel = (cand_v == m[:, None]) & (cand_c == a[:, None])
            cand_v = jnp.where(sel, -jnp.inf, cand_v)
            m = jnp.max(cand_v, axis=1)

    _store_topk(temp, k, rows, vals, idxs, w_ref, idx_ref)

    bad = jnp.any(spill >= vals[:, k - 1][:, None])

    @pl.when(bad)
    def _slow():
        s2 = sim
        vals2 = jnp.zeros((rows, k), jnp.float32)
        idxs2 = jnp.zeros((rows, k), jnp.int32)
        m2 = jnp.max(s2, axis=1)
        for j in range(k):
            a2 = jnp.min(jnp.where(s2 == m2[:, None], col, cpad), axis=1)
            vals2 = jnp.where(lane == j, m2[:, None], vals2)
            idxs2 = jnp.where(lane == j, a2[:, None], idxs2)
            if j < k - 1:
                s3 = jnp.where(col == a2[:, None], -jnp.inf, s2)
                m2 = jnp.max(s3, axis=1)
                s2 = s3
        _store_topk(temp, k, rows, vals2, idxs2, w_ref, idx_ref)


def _graph_weights(e):
    et = jnp.zeros((_D, _CPAD), jnp.float32).at[:, :_N].set(e.T)
    grid = _N // _ROWS
    return pl.pallas_call(
        functools.partial(_topk_softmax_body, _N, _K, _TEMP, _ROWS, _CPAD,
                          _SLOTS),
        grid=(grid,),
        in_specs=[
            pl.BlockSpec((_ROWS, _D), lambda i: (i, 0)),
            pl.BlockSpec((_D, _CPAD), lambda i: (0, 0)),
        ],
        out_specs=[
            pl.BlockSpec((_ROWS, _K), lambda i: (i, 0)),
            pl.BlockSpec((_ROWS, _K), lambda i: (i, 0)),
        ],
        out_shape=[
            jax.ShapeDtypeStruct((_N, _K), jnp.float32),
            jax.ShapeDtypeStruct((_N, _K), jnp.int32),
        ],
        compiler_params=pltpu.CompilerParams(
            dimension_semantics=("arbitrary",),
        ),
    )(e, et)


def kernel(node_emb, step, detach_weights):
    nrm = jnp.sqrt(jnp.sum(node_emb * node_emb, axis=1, keepdims=True))
    e = node_emb / jnp.maximum(nrm, _EPS)
    w, knn_idx = _graph_weights(e)

    row = jnp.repeat(jnp.arange(_N, dtype=jnp.int64), _K)
    col = knn_idx.reshape(-1).astype(jnp.int64)
    val = w.reshape(-1)

    self_row = jnp.arange(_N, dtype=jnp.int64)
    self_val = jnp.full((_N,), 1.0 / (_K + 1), dtype=val.dtype)
    row = jnp.concatenate([row, self_row], axis=0)
    col = jnp.concatenate([col, self_row], axis=0)
    val = jnp.concatenate([val, self_val], axis=0)

    row0, col0, val0 = row, col, val
    row = jnp.concatenate([row0, col0], axis=0)
    col = jnp.concatenate([col0, row0], axis=0)
    val = jnp.concatenate([val0, val0], axis=0)

    indices = jnp.stack([row, col], axis=0)
    return indices, val


# speedup vs baseline: 10.0548x; 1.0510x over previous
import functools

import jax
import jax.numpy as jnp
from jax.experimental import pallas as pl
from jax.experimental.pallas import tpu as pltpu

_N = 10000
_D = 128
_K = 16
_TEMP = 0.07
_EPS = 1e-8
_ROWS = 400
_CPAD = 10240
_SLOTS = 4


def _store_topk(temp, k, rows, vals, idxs, w_ref, idx_ref):
    ex = jnp.exp((vals - vals[:, 0][:, None]) / temp)
    s = jnp.sum(ex, axis=1, keepdims=True)
    w_ref[...] = ex / s * (k / (k + 1.0))
    idx_ref[...] = idxs


def _topk_softmax_body(n, k, temp, rows, cpad, slots, e_ref, et_ref, w_ref,
                       idx_ref):
    pid = pl.program_id(0)
    sim = jnp.dot(e_ref[...], et_ref[...], preferred_element_type=jnp.float32)
    row_g = pid * rows + jax.lax.broadcasted_iota(jnp.int32, (rows, cpad), 0)
    col = jax.lax.broadcasted_iota(jnp.int32, (rows, cpad), 1)
    sim = jnp.where((col == row_g) | (col >= n), -1e9, sim)

    nb = cpad // 128
    lane128 = jax.lax.broadcasted_iota(jnp.int32, (rows, 128), 1)
    neg = jnp.full((rows, 128), -jnp.inf, jnp.float32)
    rv = [neg for _ in range(slots)]
    rb = [jnp.zeros((rows, 128), jnp.int32) for _ in range(slots)]
    spill = neg
    los_v, los_b = neg, jnp.zeros((rows, 128), jnp.int32)
    for p in range(nb // 2):
        b0, b1 = 2 * p, 2 * p + 1
        va = sim[:, b0 * 128:(b0 + 1) * 128]
        vb = sim[:, b1 * 128:(b1 + 1) * 128]
        aw = va >= vb
        v = jnp.where(aw, va, vb)
        bb = jnp.where(aw, b0, b1).astype(jnp.int32)
        lv = jnp.where(aw, vb, va)
        lb = jnp.where(aw, b1, b0).astype(jnp.int32)
        for s in range(slots):
            keep = rv[s] >= v
            nv = jnp.where(keep, rv[s], v)
            nbk = jnp.where(keep, rb[s], bb)
            v = jnp.where(keep, v, rv[s])
            if s < slots - 1:
                bb = jnp.where(keep, bb, rb[s])
            rv[s], rb[s] = nv, nbk
        spill = jnp.maximum(spill, v)
        lkeep = los_v >= lv
        spill = jnp.maximum(spill, jnp.where(lkeep, lv, los_v))
        los_b = jnp.where(lkeep, los_b, lb)
        los_v = jnp.where(lkeep, los_v, lv)

    cand_v = jnp.concatenate(rv + [los_v], axis=1)
    cand_c = jnp.concatenate([r * 128 + lane128 for r in rb + [los_b]],
                             axis=1)

    lane = jax.lax.broadcasted_iota(jnp.int32, (rows, k), 1)
    vals = jnp.zeros((rows, k), jnp.float32)
    idxs = jnp.zeros((rows, k), jnp.int32)
    m = jnp.max(cand_v, axis=1)
    for j in range(k):
        a = jnp.min(jnp.where(cand_v == m[:, None], cand_c, cpad), axis=1)
        vals = jnp.where(lane == j, m[:, None], vals)
        idxs = jnp.where(lane == j, a[:, None], idxs)
        if j < k - 1:
            sel = (cand_v == m[:, None]) & (cand_c == a[:, None])
            cand_v = jnp.where(sel, -jnp.inf, cand_v)
            m = jnp.max(cand_v, axis=1)

    _store_topk(temp, k, rows, vals, idxs, w_ref, idx_ref)

    bad = jnp.any(spill >= vals[:, k - 1][:, None])

    @pl.when(bad)
    def _slow():
        s2 = sim
        vals2 = jnp.zeros((rows, k), jnp.float32)
        idxs2 = jnp.zeros((rows, k), jnp.int32)
        m2 = jnp.max(s2, axis=1)
        for j in range(k):
            a2 = jnp.min(jnp.where(s2 == m2[:, None], col, cpad), axis=1)
            vals2 = jnp.where(lane == j, m2[:, None], vals2)
            idxs2 = jnp.where(lane == j, a2[:, None], idxs2)
            if j < k - 1:
                s3 = jnp.where(col == a2[:, None], -jnp.inf, s2)
                m2 = jnp.max(s3, axis=1)
                s2 = s3
        _store_topk(temp, k, rows, vals2, idxs2, w_ref, idx_ref)


def _graph_weights(e):
    et = jnp.zeros((_D, _CPAD), jnp.float32).at[:, :_N].set(e.T)
    grid = _N // _ROWS
    return pl.pallas_call(
        functools.partial(_topk_softmax_body, _N, _K, _TEMP, _ROWS, _CPAD,
                          _SLOTS),
        grid=(grid,),
        in_specs=[
            pl.BlockSpec((_ROWS, _D), lambda i: (i, 0)),
            pl.BlockSpec((_D, _CPAD), lambda i: (0, 0)),
        ],
        out_specs=[
            pl.BlockSpec((_ROWS, _K), lambda i: (i, 0)),
            pl.BlockSpec((_ROWS, _K), lambda i: (i, 0)),
        ],
        out_shape=[
            jax.ShapeDtypeStruct((_N, _K), jnp.float32),
            jax.ShapeDtypeStruct((_N, _K), jnp.int32),
        ],
        compiler_params=pltpu.CompilerParams(
            dimension_semantics=("arbitrary",),
        ),
    )(e, et)


def kernel(node_emb, step, detach_weights):
    nrm = jnp.sqrt(jnp.sum(node_emb * node_emb, axis=1, keepdims=True))
    e = node_emb / jnp.maximum(nrm, _EPS)
    w, knn_idx = _graph_weights(e)

    row = jnp.repeat(jnp.arange(_N, dtype=jnp.int64), _K)
    col = knn_idx.reshape(-1).astype(jnp.int64)
    val = w.reshape(-1)

    self_row = jnp.arange(_N, dtype=jnp.int64)
    self_val = jnp.full((_N,), 1.0 / (_K + 1), dtype=val.dtype)
    row = jnp.concatenate([row, self_row], axis=0)
    col = jnp.concatenate([col, self_row], axis=0)
    val = jnp.concatenate([val, self_val], axis=0)

    row0, col0, val0 = row, col, val
    row = jnp.concatenate([row0, col0], axis=0)
    col = jnp.concatenate([col0, row0], axis=0)
    val = jnp.concatenate([val0, val0], axis=0)

    indices = jnp.stack([row, col], axis=0)
    return indices, val


# speedup vs baseline: 10.0979x; 1.0043x over previous
import functools

import jax
import jax.numpy as jnp
from jax.experimental import pallas as pl
from jax.experimental.pallas import tpu as pltpu

_N = 10000
_D = 128
_K = 16
_TEMP = 0.07
_EPS = 1e-8
_ROWS = 400
_CPAD = 10240
_SLOTS = 4


def _store_topk(temp, k, rows, vals, idxs, w_ref, idx_ref):
    ex = jnp.exp((vals - vals[:, 0][:, None]) / temp)
    s = jnp.sum(ex, axis=1, keepdims=True)
    w_ref[...] = ex / s * (k / (k + 1.0))
    idx_ref[...] = idxs


def _topk_softmax_body(n, k, temp, rows, cpad, slots, e_ref, et_ref, w_ref,
                       idx_ref):
    pid = pl.program_id(0)
    sim = jnp.dot(e_ref[...], et_ref[...], preferred_element_type=jnp.float32)
    row_g = pid * rows + jax.lax.broadcasted_iota(jnp.int32, (rows, cpad), 0)
    col = jax.lax.broadcasted_iota(jnp.int32, (rows, cpad), 1)
    sim = jnp.where((col == row_g) | (col >= n), -1e9, sim)

    nb = cpad // 128
    lane128 = jax.lax.broadcasted_iota(jnp.int32, (rows, 128), 1)
    neg = jnp.full((rows, 128), -jnp.inf, jnp.float32)
    rv = [neg for _ in range(slots)]
    rb = [jnp.zeros((rows, 128), jnp.int32) for _ in range(slots)]
    spill = neg
    los_v, los_b = neg, jnp.zeros((rows, 128), jnp.int32)
    for p in range(nb // 2):
        b0, b1 = 2 * p, 2 * p + 1
        va = sim[:, b0 * 128:(b0 + 1) * 128]
        vb = sim[:, b1 * 128:(b1 + 1) * 128]
        aw = va >= vb
        v = jnp.where(aw, va, vb)
        bb = jnp.where(aw, b0, b1).astype(jnp.int32)
        lv = jnp.where(aw, vb, va)
        lb = jnp.where(aw, b1, b0).astype(jnp.int32)
        for s in range(slots):
            keep = rv[s] >= v
            nv = jnp.where(keep, rv[s], v)
            nbk = jnp.where(keep, rb[s], bb)
            v = jnp.where(keep, v, rv[s])
            if s < slots - 1:
                bb = jnp.where(keep, bb, rb[s])
            rv[s], rb[s] = nv, nbk
        spill = jnp.maximum(spill, v)
        lkeep = los_v >= lv
        spill = jnp.maximum(spill, jnp.where(lkeep, lv, los_v))
        los_b = jnp.where(lkeep, los_b, lb)
        los_v = jnp.where(lkeep, los_v, lv)

    v, bb = los_v, los_b
    for s in range(slots):
        keep = rv[s] >= v
        nv = jnp.where(keep, rv[s], v)
        nbk = jnp.where(keep, rb[s], bb)
        v = jnp.where(keep, v, rv[s])
        bb = jnp.where(keep, bb, rb[s])
        rv[s], rb[s] = nv, nbk
    rv.append(v)
    rb.append(bb)
    depth = slots + 1
    for _ in range(depth - 1):
        for s in range(depth - 1):
            swap = (rv[s] == rv[s + 1]) & (rb[s] > rb[s + 1])
            lo = jnp.where(swap, rb[s + 1], rb[s])
            hi = jnp.where(swap, rb[s], rb[s + 1])
            rb[s], rb[s + 1] = lo, hi
    rc = [r * 128 + lane128 for r in rb]

    lane = jax.lax.broadcasted_iota(jnp.int32, (rows, k), 1)
    vals = jnp.zeros((rows, k), jnp.float32)
    idxs = jnp.zeros((rows, k), jnp.int32)
    big = jnp.full((rows, 128), cpad, jnp.int32)
    for j in range(k):
        m = jnp.max(rv[0], axis=1)
        a = jnp.min(jnp.where(rv[0] == m[:, None], rc[0], big), axis=1)
        vals = jnp.where(lane == j, m[:, None], vals)
        idxs = jnp.where(lane == j, a[:, None], idxs)
        if j < k - 1:
            sel = (rv[0] == m[:, None]) & (rc[0] == a[:, None])
            for s in range(depth - 1):
                rv[s] = jnp.where(sel, rv[s + 1], rv[s])
                rc[s] = jnp.where(sel, rc[s + 1], rc[s])
            rv[depth - 1] = jnp.where(sel, -jnp.inf, rv[depth - 1])

    _store_topk(temp, k, rows, vals, idxs, w_ref, idx_ref)

    bad = jnp.any(spill >= vals[:, k - 1][:, None])

    @pl.when(bad)
    def _slow():
        s2 = sim
        vals2 = jnp.zeros((rows, k), jnp.float32)
        idxs2 = jnp.zeros((rows, k), jnp.int32)
        m2 = jnp.max(s2, axis=1)
        for j in range(k):
            a2 = jnp.min(jnp.where(s2 == m2[:, None], col, cpad), axis=1)
            vals2 = jnp.where(lane == j, m2[:, None], vals2)
            idxs2 = jnp.where(lane == j, a2[:, None], idxs2)
            if j < k - 1:
                s3 = jnp.where(col == a2[:, None], -jnp.inf, s2)
                m2 = jnp.max(s3, axis=1)
                s2 = s3
        _store_topk(temp, k, rows, vals2, idxs2, w_ref, idx_ref)


def _graph_weights(e):
    et = jnp.zeros((_D, _CPAD), jnp.float32).at[:, :_N].set(e.T)
    grid = _N // _ROWS
    return pl.pallas_call(
        functools.partial(_topk_softmax_body, _N, _K, _TEMP, _ROWS, _CPAD,
                          _SLOTS),
        grid=(grid,),
        in_specs=[
            pl.BlockSpec((_ROWS, _D), lambda i: (i, 0)),
            pl.BlockSpec((_D, _CPAD), lambda i: (0, 0)),
        ],
        out_specs=[
            pl.BlockSpec((_ROWS, _K), lambda i: (i, 0)),
            pl.BlockSpec((_ROWS, _K), lambda i: (i, 0)),
        ],
        out_shape=[
            jax.ShapeDtypeStruct((_N, _K), jnp.float32),
            jax.ShapeDtypeStruct((_N, _K), jnp.int32),
        ],
        compiler_params=pltpu.CompilerParams(
            dimension_semantics=("arbitrary",),
        ),
    )(e, et)


def kernel(node_emb, step, detach_weights):
    nrm = jnp.sqrt(jnp.sum(node_emb * node_emb, axis=1, keepdims=True))
    e = node_emb / jnp.maximum(nrm, _EPS)
    w, knn_idx = _graph_weights(e)

    row = jnp.repeat(jnp.arange(_N, dtype=jnp.int64), _K)
    col = knn_idx.reshape(-1).astype(jnp.int64)
    val = w.reshape(-1)

    self_row = jnp.arange(_N, dtype=jnp.int64)
    self_val = jnp.full((_N,), 1.0 / (_K + 1), dtype=val.dtype)
    row = jnp.concatenate([row, self_row], axis=0)
    col = jnp.concatenate([col, self_row], axis=0)
    val = jnp.concatenate([val, self_val], axis=0)

    row0, col0, val0 = row, col, val
    row = jnp.concatenate([row0, col0], axis=0)
    col = jnp.concatenate([col0, row0], axis=0)
    val = jnp.concatenate([val0, val0], axis=0)

    indices = jnp.stack([row, col], axis=0)
    return indices, val
